# baseline (speedup 1.0000x reference)
import functools

import jax
import jax.numpy as jnp
from jax import lax
from jax.experimental import pallas as pl
from jax.experimental.pallas import tpu as pltpu
from jax.experimental.pallas import tpu_sc as plsc

TOPK = 2
BLK = 512


def _routing_body(x_ref, gw_ref, p_ref, w_ref, meta_ref, *, T, E, NT):
    x = x_ref[...]
    gw = gw_ref[...]
    logits = lax.dot_general(x, gw, (((1,), (1,)), ((), ())),
                             preferred_element_type=jnp.float32)
    m = jnp.max(logits, axis=1, keepdims=True)
    ex = jnp.exp(logits - m)
    probs = ex / jnp.sum(ex, axis=1, keepdims=True)

    iota_e = lax.broadcasted_iota(jnp.int32, (T, E), 1)
    v0 = jnp.max(probs, axis=1, keepdims=True)
    i0 = jnp.min(jnp.where(probs == v0, iota_e, E), axis=1, keepdims=True)
    sel0 = iota_e == i0
    probs1 = jnp.where(sel0, -1.0, probs)
    v1 = jnp.max(probs1, axis=1, keepdims=True)
    i1 = jnp.min(jnp.where(probs1 == v1, iota_e, E), axis=1, keepdims=True)
    sel1 = iota_e == i1
    s = v0 + v1
    w0 = v0 / s
    w1 = v1 / s

    N = TOPK * T
    A = jnp.concatenate([sel0, sel1], axis=0).astype(jnp.float32)
    At = jnp.transpose(A, (1, 0))
    Ct = At
    sft = 1
    while sft < N:
        Ct = Ct + jnp.concatenate(
            [jnp.zeros((E, sft), jnp.float32), Ct[:, : N - sft]], axis=1)
        sft *= 2
    counts_c = Ct[:, N - 1: N]
    fblk = jnp.float32(BLK)
    pc_c = jnp.floor((counts_c + (fblk - 1.0)) * (1.0 / fblk)) * fblk
    ii = lax.broadcasted_iota(jnp.int32, (E, E), 0)
    ee = lax.broadcasted_iota(jnp.int32, (E, E), 1)
    lower_t = (ee < ii).astype(jnp.float32)
    base_c = lax.dot_general(lower_t, pc_c, (((1,), (0,)), ((), ())))
    slot_t = jnp.sum(At * (Ct - 1.0 + base_c), axis=0, keepdims=True)
    slot_t = slot_t.astype(jnp.int32)
    s0 = jnp.transpose(slot_t[:, :T], (1, 0))
    s1 = jnp.transpose(slot_t[:, T:], (1, 0))

    col = lax.broadcasted_iota(jnp.int32, (T, E), 1)
    p_ref[...] = jnp.where(col == 0, s0, jnp.where(col == 1, s1, 0))
    w_ref[...] = jnp.where(col == 0, w0, jnp.where(col == 1, w1, 0.0))

    base = jnp.transpose(base_c, (1, 0))
    pc = jnp.transpose(pc_c, (1, 0))
    counts = jnp.transpose(counts_c, (1, 0))
    ti = lax.broadcasted_iota(jnp.int32, (NT, E), 0)
    te = lax.broadcasted_iota(jnp.int32, (NT, E), 1)
    row = jnp.float32(BLK) * ti.astype(jnp.float32)
    inside = (row >= base) & (row < base + pc)
    eid = jnp.sum(jnp.where(inside, te, 0), axis=1, keepdims=True)
    nt_used = jnp.sum(pc_c) * (1.0 / fblk)
    tif = ti[:, :1].astype(jnp.float32)
    flag = (tif < nt_used).astype(jnp.int32)
    emax = jnp.max(jnp.where(counts > 0.0, ee[:1, :], 0), axis=1, keepdims=True)
    eid = jnp.where(flag == 1, eid, emax)
    xdi = jnp.minimum(tif, nt_used - 1.0).astype(jnp.int32)
    mcol = lax.broadcasted_iota(jnp.int32, (NT, E), 1)
    meta_ref[...] = jnp.where(
        mcol == 0, eid,
        jnp.where(mcol == 1, flag, jnp.where(mcol == 2, xdi, 0)))


def _routing(x, gate_w, NT):
    T, D = x.shape
    E = gate_w.shape[0]
    body = functools.partial(_routing_body, T=T, E=E, NT=NT)
    return pl.pallas_call(
        body,
        out_shape=(
            jax.ShapeDtypeStruct((T, E), jnp.int32),
            jax.ShapeDtypeStruct((T, E), jnp.float32),
            jax.ShapeDtypeStruct((NT, E), jnp.int32),
        ),
    )(x, gate_w)


def _sc_dispatch(x, slots, n_rows):
    T, D = x.shape
    N = slots.shape[0]
    info = plsc.get_sparse_core_info()
    NW = info.num_cores * info.num_subcores
    chunk = N // NW
    mesh = plsc.VectorSubcoreMesh(core_axis_name="c", subcore_axis_name="s")

    @functools.partial(
        pl.kernel,
        mesh=mesh,
        out_type=jax.ShapeDtypeStruct((n_rows, D), jnp.float32),
        scratch_types=[
            pltpu.VMEM((chunk,), jnp.int32),
            pltpu.VMEM((chunk, D), jnp.float32),
            pltpu.SemaphoreType.DMA,
        ],
    )
    def k(x_hbm, slots_hbm, out_hbm, idx_v, rows_v, sem):
        wid = lax.axis_index("s") * info.num_cores + lax.axis_index("c")
        jbase = wid * chunk
        tbase = jnp.where(jbase >= T, jbase - T, jbase)
        pltpu.sync_copy(slots_hbm.at[pl.ds(jbase, chunk)], idx_v)
        pltpu.sync_copy(x_hbm.at[pl.ds(tbase, chunk)], rows_v)
        pltpu.async_copy(rows_v, out_hbm.at[idx_v], sem).wait()

    return k(x, slots)


def _sc_combine(y, slots):
    R, D = y.shape
    N = slots.shape[0]
    info = plsc.get_sparse_core_info()
    NW = info.num_cores * info.num_subcores
    chunk = N // NW
    mesh = plsc.VectorSubcoreMesh(core_axis_name="c", subcore_axis_name="s")

    @functools.partial(
        pl.kernel,
        mesh=mesh,
        out_type=jax.ShapeDtypeStruct((N, D), jnp.float32),
        scratch_types=[
            pltpu.VMEM((chunk,), jnp.int32),
            pltpu.VMEM((chunk, D), jnp.float32),
            pltpu.SemaphoreType.DMA,
        ],
    )
    def k(y_hbm, slots_hbm, out_hbm, idx_v, rows_v, sem):
        wid = lax.axis_index("s") * info.num_cores + lax.axis_index("c")
        jbase = wid * chunk
        pltpu.sync_copy(slots_hbm.at[pl.ds(jbase, chunk)], idx_v)
        pltpu.async_copy(y_hbm.at[idx_v], rows_v, sem).wait()
        pltpu.sync_copy(rows_v, out_hbm.at[pl.ds(jbase, chunk)])

    return k(y, slots)


def _ffn_body(eid_ref, flag_ref, xdi_ref, xd_ref, wg_ref, wu_ref, wd_ref,
              y_ref):
    i = pl.program_id(0)

    @pl.when(flag_ref[i] == 1)
    def _():
        xt = xd_ref[...]
        g = jnp.dot(xt, wg_ref[0], preferred_element_type=jnp.float32)
        u = jnp.dot(xt, wu_ref[0], preferred_element_type=jnp.float32)
        h = g * jax.nn.sigmoid(g) * u
        y_ref[...] = jnp.dot(h, wd_ref[0], preferred_element_type=jnp.float32)


def _grouped_ffn(eid, flag, xdi, xd, w_gate, w_up, w_down, NT):
    R, D = xd.shape
    FF = w_gate.shape[2]
    grid_spec = pltpu.PrefetchScalarGridSpec(
        num_scalar_prefetch=3,
        grid=(NT,),
        in_specs=[
            pl.BlockSpec((BLK, D), lambda i, e, f, xi: (xi[i], 0)),
            pl.BlockSpec((1, D, FF), lambda i, e, f, xi: (e[i], 0, 0)),
            pl.BlockSpec((1, D, FF), lambda i, e, f, xi: (e[i], 0, 0)),
            pl.BlockSpec((1, FF, D), lambda i, e, f, xi: (e[i], 0, 0)),
        ],
        out_specs=pl.BlockSpec((BLK, D), lambda i, e, f, xi: (xi[i], 0)),
    )
    return pl.pallas_call(
        _ffn_body,
        grid_spec=grid_spec,
        out_shape=jax.ShapeDtypeStruct((R, D), jnp.float32),
    )(eid, flag, xdi, xd, w_gate, w_up, w_down)


def _shared_body(x_ref, sg_ref, su_ref, sd_ref, segw_ref, o_ref):
    x = x_ref[...]
    g = jnp.dot(x, sg_ref[...], preferred_element_type=jnp.float32)
    u = jnp.dot(x, su_ref[...], preferred_element_type=jnp.float32)
    h = g * jax.nn.sigmoid(g) * u
    sh = jnp.dot(h, sd_ref[...], preferred_element_type=jnp.float32)
    gate = jax.nn.sigmoid(jnp.dot(x, segw_ref[...],
                                  preferred_element_type=jnp.float32))[:, 0:1]
    o_ref[...] = gate * sh


def _shared(x, s_gate, s_up, s_down, segw_p):
    T, D = x.shape
    FFS = s_gate.shape[1]
    TB = 512
    return pl.pallas_call(
        _shared_body,
        grid=(T // TB,),
        in_specs=[
            pl.BlockSpec((TB, D), lambda i: (i, 0)),
            pl.BlockSpec((D, FFS), lambda i: (0, 0)),
            pl.BlockSpec((D, FFS), lambda i: (0, 0)),
            pl.BlockSpec((FFS, D), lambda i: (0, 0)),
            pl.BlockSpec((D, 128), lambda i: (0, 0)),
        ],
        out_specs=pl.BlockSpec((TB, D), lambda i: (i, 0)),
        out_shape=jax.ShapeDtypeStruct((T, D), jnp.float32),
    )(x, s_gate, s_up, s_down, segw_p)


def _final_body(y0_ref, y1_ref, w_ref, sh_ref, o_ref):
    w0 = w_ref[:, 0:1]
    w1 = w_ref[:, 1:2]
    o_ref[...] = w0 * y0_ref[...] + w1 * y1_ref[...] + sh_ref[...]


def _final(y0, y1, wts, sh):
    T, D = y0.shape
    E = wts.shape[1]
    TB = 512
    return pl.pallas_call(
        _final_body,
        grid=(T // TB,),
        in_specs=[
            pl.BlockSpec((TB, D), lambda i: (i, 0)),
            pl.BlockSpec((TB, D), lambda i: (i, 0)),
            pl.BlockSpec((TB, E), lambda i: (i, 0)),
            pl.BlockSpec((TB, D), lambda i: (i, 0)),
        ],
        out_specs=pl.BlockSpec((TB, D), lambda i: (i, 0)),
        out_shape=jax.ShapeDtypeStruct((T, D), jnp.float32),
    )(y0, y1, wts, sh)


def kernel(hidden_states, gate_w, w_gate, w_up, w_down, s_gate, s_up, s_down,
           seg_w):
    orig_shape = hidden_states.shape
    D = orig_shape[-1]
    x = hidden_states.reshape(-1, D)
    T = x.shape[0]
    E = w_gate.shape[0]
    NT = (T * TOPK) // BLK + E
    n_rows = NT * BLK

    slots2, wts, meta = _routing(x, gate_w, NT)
    slots = jnp.concatenate([slots2[:, 0], slots2[:, 1]])
    eid = meta[:, 0]
    flag = meta[:, 1]
    xdi = meta[:, 2]

    xd = _sc_dispatch(x, slots, n_rows)
    segw_p = jnp.pad(seg_w, ((0, 0), (0, 128 - seg_w.shape[1])))
    sh = _shared(x, s_gate, s_up, s_down, segw_p)
    y = _grouped_ffn(eid, flag, xdi, xd, w_gate, w_up, w_down, NT)
    yg = _sc_combine(y, slots)
    y0 = yg[:T]
    y1 = yg[T:]

    out = _final(y0, y1, wts, sh)
    return out.reshape(orig_shape)


# speedup vs baseline: 1.1013x; 1.1013x over previous
import functools

import jax
import jax.numpy as jnp
from jax import lax
from jax.experimental import pallas as pl
from jax.experimental.pallas import tpu as pltpu
from jax.experimental.pallas import tpu_sc as plsc

TOPK = 2
BLK = 640


def _routing_body(x_ref, gw_ref, p_ref, w_ref, meta_ref, *, T, E, NT):
    x = x_ref[...]
    gw = gw_ref[...]
    logits = lax.dot_general(x, gw, (((1,), (1,)), ((), ())),
                             preferred_element_type=jnp.float32)
    m = jnp.max(logits, axis=1, keepdims=True)
    ex = jnp.exp(logits - m)
    probs = ex / jnp.sum(ex, axis=1, keepdims=True)

    iota_e = lax.broadcasted_iota(jnp.int32, (T, E), 1)
    v0 = jnp.max(probs, axis=1, keepdims=True)
    i0 = jnp.min(jnp.where(probs == v0, iota_e, E), axis=1, keepdims=True)
    sel0 = iota_e == i0
    probs1 = jnp.where(sel0, -1.0, probs)
    v1 = jnp.max(probs1, axis=1, keepdims=True)
    i1 = jnp.min(jnp.where(probs1 == v1, iota_e, E), axis=1, keepdims=True)
    sel1 = iota_e == i1
    s = v0 + v1
    w0 = v0 / s
    w1 = v1 / s

    N = TOPK * T
    A = jnp.concatenate([sel0, sel1], axis=0).astype(jnp.float32)
    At = jnp.transpose(A, (1, 0))
    Ct = At
    sft = 1
    while sft < N:
        Ct = Ct + jnp.concatenate(
            [jnp.zeros((E, sft), jnp.float32), Ct[:, : N - sft]], axis=1)
        sft *= 2
    counts_c = Ct[:, N - 1: N]
    fblk = jnp.float32(BLK)
    pc_c = jnp.floor((counts_c + (fblk - 1.0)) * (1.0 / fblk)) * fblk
    ii = lax.broadcasted_iota(jnp.int32, (E, E), 0)
    ee = lax.broadcasted_iota(jnp.int32, (E, E), 1)
    lower_t = (ee < ii).astype(jnp.float32)
    base_c = lax.dot_general(lower_t, pc_c, (((1,), (0,)), ((), ())))
    slot_t = jnp.sum(At * (Ct - 1.0 + base_c), axis=0, keepdims=True)
    slot_t = slot_t.astype(jnp.int32)
    s0 = jnp.transpose(slot_t[:, :T], (1, 0))
    s1 = jnp.transpose(slot_t[:, T:], (1, 0))

    col = lax.broadcasted_iota(jnp.int32, (T, E), 1)
    p_ref[...] = jnp.where(col == 0, s0, jnp.where(col == 1, s1, 0))
    w_ref[...] = jnp.where(col == 0, w0, jnp.where(col == 1, w1, 0.0))

    base = jnp.transpose(base_c, (1, 0))
    pc = jnp.transpose(pc_c, (1, 0))
    counts = jnp.transpose(counts_c, (1, 0))
    ti = lax.broadcasted_iota(jnp.int32, (NT, E), 0)
    te = lax.broadcasted_iota(jnp.int32, (NT, E), 1)
    row = jnp.float32(BLK) * ti.astype(jnp.float32)
    inside = (row >= base) & (row < base + pc)
    eid = jnp.sum(jnp.where(inside, te, 0), axis=1, keepdims=True)
    nt_used = jnp.sum(pc_c) * (1.0 / fblk)
    tif = ti[:, :1].astype(jnp.float32)
    flag = (tif < nt_used).astype(jnp.int32)
    emax = jnp.max(jnp.where(counts > 0.0, ee[:1, :], 0), axis=1, keepdims=True)
    eid = jnp.where(flag == 1, eid, emax)
    xdi = jnp.minimum(tif, nt_used - 1.0).astype(jnp.int32)
    mcol = lax.broadcasted_iota(jnp.int32, (NT, E), 1)
    meta_ref[...] = jnp.where(
        mcol == 0, eid,
        jnp.where(mcol == 1, flag, jnp.where(mcol == 2, xdi, 0)))


def _routing(x, gate_w, NT):
    T, D = x.shape
    E = gate_w.shape[0]
    body = functools.partial(_routing_body, T=T, E=E, NT=NT)
    return pl.pallas_call(
        body,
        out_shape=(
            jax.ShapeDtypeStruct((T, E), jnp.int32),
            jax.ShapeDtypeStruct((T, E), jnp.float32),
            jax.ShapeDtypeStruct((NT, E), jnp.int32),
        ),
    )(x, gate_w)


def _sc_dispatch(x, slots, n_rows):
    T, D = x.shape
    N = slots.shape[0]
    info = plsc.get_sparse_core_info()
    NW = info.num_cores * info.num_subcores
    chunk = N // NW
    mesh = plsc.VectorSubcoreMesh(core_axis_name="c", subcore_axis_name="s")

    @functools.partial(
        pl.kernel,
        mesh=mesh,
        out_type=jax.ShapeDtypeStruct((n_rows, D), jnp.float32),
        scratch_types=[
            pltpu.VMEM((chunk,), jnp.int32),
            pltpu.VMEM((chunk, D), jnp.float32),
            pltpu.SemaphoreType.DMA,
        ],
    )
    def k(x_hbm, slots_hbm, out_hbm, idx_v, rows_v, sem):
        wid = lax.axis_index("s") * info.num_cores + lax.axis_index("c")
        jbase = wid * chunk
        tbase = jnp.where(jbase >= T, jbase - T, jbase)
        pltpu.sync_copy(slots_hbm.at[pl.ds(jbase, chunk)], idx_v)
        pltpu.sync_copy(x_hbm.at[pl.ds(tbase, chunk)], rows_v)
        pltpu.async_copy(rows_v, out_hbm.at[idx_v], sem).wait()

    return k(x, slots)


def _sc_combine(y, slots):
    R, D = y.shape
    N = slots.shape[0]
    info = plsc.get_sparse_core_info()
    NW = info.num_cores * info.num_subcores
    chunk = N // NW
    mesh = plsc.VectorSubcoreMesh(core_axis_name="c", subcore_axis_name="s")

    @functools.partial(
        pl.kernel,
        mesh=mesh,
        out_type=jax.ShapeDtypeStruct((N, D), jnp.float32),
        scratch_types=[
            pltpu.VMEM((chunk,), jnp.int32),
            pltpu.VMEM((chunk, D), jnp.float32),
            pltpu.SemaphoreType.DMA,
        ],
    )
    def k(y_hbm, slots_hbm, out_hbm, idx_v, rows_v, sem):
        wid = lax.axis_index("s") * info.num_cores + lax.axis_index("c")
        jbase = wid * chunk
        pltpu.sync_copy(slots_hbm.at[pl.ds(jbase, chunk)], idx_v)
        pltpu.async_copy(y_hbm.at[idx_v], rows_v, sem).wait()
        pltpu.sync_copy(rows_v, out_hbm.at[pl.ds(jbase, chunk)])

    return k(y, slots)


def _ffn_body(eid_ref, flag_ref, xdi_ref, xd_ref, wg_ref, wu_ref, wd_ref,
              y_ref):
    i = pl.program_id(0)

    @pl.when(flag_ref[i] == 1)
    def _():
        xt = xd_ref[...]
        g = jnp.dot(xt, wg_ref[0], preferred_element_type=jnp.float32)
        u = jnp.dot(xt, wu_ref[0], preferred_element_type=jnp.float32)
        h = g * jax.nn.sigmoid(g) * u
        y_ref[...] = jnp.dot(h, wd_ref[0], preferred_element_type=jnp.float32)


def _grouped_ffn(eid, flag, xdi, xd, w_gate, w_up, w_down, NT):
    R, D = xd.shape
    FF = w_gate.shape[2]
    grid_spec = pltpu.PrefetchScalarGridSpec(
        num_scalar_prefetch=3,
        grid=(NT,),
        in_specs=[
            pl.BlockSpec((BLK, D), lambda i, e, f, xi: (xi[i], 0)),
            pl.BlockSpec((1, D, FF), lambda i, e, f, xi: (e[i], 0, 0)),
            pl.BlockSpec((1, D, FF), lambda i, e, f, xi: (e[i], 0, 0)),
            pl.BlockSpec((1, FF, D), lambda i, e, f, xi: (e[i], 0, 0)),
        ],
        out_specs=pl.BlockSpec((BLK, D), lambda i, e, f, xi: (xi[i], 0)),
    )
    return pl.pallas_call(
        _ffn_body,
        grid_spec=grid_spec,
        out_shape=jax.ShapeDtypeStruct((R, D), jnp.float32),
    )(eid, flag, xdi, xd, w_gate, w_up, w_down)


def _final_body(x_ref, y0_ref, y1_ref, w_ref, sg_ref, su_ref, sd_ref,
                segw_ref, o_ref):
    x = x_ref[...]
    g = jnp.dot(x, sg_ref[...], preferred_element_type=jnp.float32)
    u = jnp.dot(x, su_ref[...], preferred_element_type=jnp.float32)
    h = g * jax.nn.sigmoid(g) * u
    sh = jnp.dot(h, sd_ref[...], preferred_element_type=jnp.float32)
    gate = jax.nn.sigmoid(jnp.dot(x, segw_ref[...],
                                  preferred_element_type=jnp.float32))[:, 0:1]
    w0 = w_ref[:, 0:1]
    w1 = w_ref[:, 1:2]
    o_ref[...] = w0 * y0_ref[...] + w1 * y1_ref[...] + gate * sh


def _final(x, y0, y1, wts, s_gate, s_up, s_down, segw_p):
    T, D = x.shape
    FFS = s_gate.shape[1]
    E = wts.shape[1]
    TB = 1024
    return pl.pallas_call(
        _final_body,
        grid=(T // TB,),
        in_specs=[
            pl.BlockSpec((TB, D), lambda i: (i, 0)),
            pl.BlockSpec((TB, D), lambda i: (i, 0)),
            pl.BlockSpec((TB, D), lambda i: (i, 0)),
            pl.BlockSpec((TB, E), lambda i: (i, 0)),
            pl.BlockSpec((D, FFS), lambda i: (0, 0)),
            pl.BlockSpec((D, FFS), lambda i: (0, 0)),
            pl.BlockSpec((FFS, D), lambda i: (0, 0)),
            pl.BlockSpec((D, 128), lambda i: (0, 0)),
        ],
        out_specs=pl.BlockSpec((TB, D), lambda i: (i, 0)),
        out_shape=jax.ShapeDtypeStruct((T, D), jnp.float32),
    )(x, y0, y1, wts, s_gate, s_up, s_down, segw_p)


def kernel(hidden_states, gate_w, w_gate, w_up, w_down, s_gate, s_up, s_down,
           seg_w):
    orig_shape = hidden_states.shape
    D = orig_shape[-1]
    x = hidden_states.reshape(-1, D)
    T = x.shape[0]
    E = w_gate.shape[0]
    NT = (T * TOPK) // BLK + E
    n_rows = NT * BLK

    slots2, wts, meta = _routing(x, gate_w, NT)
    slots = jnp.concatenate([slots2[:, 0], slots2[:, 1]])
    eid = meta[:, 0]
    flag = meta[:, 1]
    xdi = meta[:, 2]

    xd = _sc_dispatch(x, slots, n_rows)
    y = _grouped_ffn(eid, flag, xdi, xd, w_gate, w_up, w_down, NT)
    yg = _sc_combine(y, slots)
    y0 = yg[:T]
    y1 = yg[T:]

    segw_p = jnp.pad(seg_w, ((0, 0), (0, 128 - seg_w.shape[1])))
    out = _final(x, y0, y1, wts, s_gate, s_up, s_down, segw_p)
    return out.reshape(orig_shape)


# speedup vs baseline: 1.1125x; 1.0102x over previous
import functools

import jax
import jax.numpy as jnp
from jax import lax
from jax.experimental import pallas as pl
from jax.experimental.pallas import tpu as pltpu
from jax.experimental.pallas import tpu_sc as plsc

TOPK = 2
BLK = 576


def _routing_body(x_ref, gw_ref, p_ref, w_ref, meta_ref, *, T, E, NT):
    x = x_ref[...]
    gw = gw_ref[...]
    logits = lax.dot_general(x, gw, (((1,), (1,)), ((), ())),
                             preferred_element_type=jnp.float32)
    m = jnp.max(logits, axis=1, keepdims=True)
    ex = jnp.exp(logits - m)
    probs = ex / jnp.sum(ex, axis=1, keepdims=True)

    iota_e = lax.broadcasted_iota(jnp.int32, (T, E), 1)
    v0 = jnp.max(probs, axis=1, keepdims=True)
    i0 = jnp.min(jnp.where(probs == v0, iota_e, E), axis=1, keepdims=True)
    sel0 = iota_e == i0
    probs1 = jnp.where(sel0, -1.0, probs)
    v1 = jnp.max(probs1, axis=1, keepdims=True)
    i1 = jnp.min(jnp.where(probs1 == v1, iota_e, E), axis=1, keepdims=True)
    sel1 = iota_e == i1
    s = v0 + v1
    w0 = v0 / s
    w1 = v1 / s

    N = TOPK * T
    A = jnp.concatenate([sel0, sel1], axis=0).astype(jnp.float32)
    At = jnp.transpose(A, (1, 0))
    Ct = At
    sft = 1
    while sft < N:
        Ct = Ct + jnp.concatenate(
            [jnp.zeros((E, sft), jnp.float32), Ct[:, : N - sft]], axis=1)
        sft *= 2
    counts_c = Ct[:, N - 1: N]
    fblk = jnp.float32(BLK)
    pc_c = jnp.floor((counts_c + (fblk - 1.0)) * (1.0 / fblk)) * fblk
    ii = lax.broadcasted_iota(jnp.int32, (E, E), 0)
    ee = lax.broadcasted_iota(jnp.int32, (E, E), 1)
    lower_t = (ee < ii).astype(jnp.float32)
    base_c = lax.dot_general(lower_t, pc_c, (((1,), (0,)), ((), ())))
    slot_t = jnp.sum(At * (Ct - 1.0 + base_c), axis=0, keepdims=True)
    slot_t = slot_t.astype(jnp.int32)
    s0 = jnp.transpose(slot_t[:, :T], (1, 0))
    s1 = jnp.transpose(slot_t[:, T:], (1, 0))

    col = lax.broadcasted_iota(jnp.int32, (T, E), 1)
    p_ref[...] = jnp.where(col == 0, s0, jnp.where(col == 1, s1, 0))
    w_ref[...] = jnp.where(col == 0, w0, jnp.where(col == 1, w1, 0.0))

    base = jnp.transpose(base_c, (1, 0))
    pc = jnp.transpose(pc_c, (1, 0))
    counts = jnp.transpose(counts_c, (1, 0))
    ti = lax.broadcasted_iota(jnp.int32, (NT, E), 0)
    te = lax.broadcasted_iota(jnp.int32, (NT, E), 1)
    row = jnp.float32(BLK) * ti.astype(jnp.float32)
    inside = (row >= base) & (row < base + pc)
    eid = jnp.sum(jnp.where(inside, te, 0), axis=1, keepdims=True)
    nt_used = jnp.sum(pc_c) * (1.0 / fblk)
    tif = ti[:, :1].astype(jnp.float32)
    flag = (tif < nt_used).astype(jnp.int32)
    emax = jnp.max(jnp.where(counts > 0.0, ee[:1, :], 0), axis=1, keepdims=True)
    eid = jnp.where(flag == 1, eid, emax)
    xdi = jnp.minimum(tif, nt_used - 1.0).astype(jnp.int32)
    mcol = lax.broadcasted_iota(jnp.int32, (NT, E), 1)
    meta_ref[...] = jnp.where(
        mcol == 0, eid,
        jnp.where(mcol == 1, flag, jnp.where(mcol == 2, xdi, 0)))


def _routing(x, gate_w, NT):
    T, D = x.shape
    E = gate_w.shape[0]
    body = functools.partial(_routing_body, T=T, E=E, NT=NT)
    return pl.pallas_call(
        body,
        out_shape=(
            jax.ShapeDtypeStruct((T, E), jnp.int32),
            jax.ShapeDtypeStruct((T, E), jnp.float32),
            jax.ShapeDtypeStruct((NT, E), jnp.int32),
        ),
    )(x, gate_w)


def _sc_dispatch(x, slots, n_rows):
    T, D = x.shape
    N = slots.shape[0]
    info = plsc.get_sparse_core_info()
    NW = info.num_cores * info.num_subcores
    chunk = N // NW
    mesh = plsc.VectorSubcoreMesh(core_axis_name="c", subcore_axis_name="s")

    @functools.partial(
        pl.kernel,
        mesh=mesh,
        out_type=jax.ShapeDtypeStruct((n_rows, D), jnp.float32),
        scratch_types=[
            pltpu.VMEM((chunk,), jnp.int32),
            pltpu.VMEM((chunk, D), jnp.float32),
            pltpu.SemaphoreType.DMA,
        ],
    )
    def k(x_hbm, slots_hbm, out_hbm, idx_v, rows_v, sem):
        wid = lax.axis_index("s") * info.num_cores + lax.axis_index("c")
        jbase = wid * chunk
        tbase = jnp.where(jbase >= T, jbase - T, jbase)
        pltpu.sync_copy(slots_hbm.at[pl.ds(jbase, chunk)], idx_v)
        pltpu.sync_copy(x_hbm.at[pl.ds(tbase, chunk)], rows_v)
        pltpu.async_copy(rows_v, out_hbm.at[idx_v], sem).wait()

    return k(x, slots)


def _sc_combine(y, slots):
    R, D = y.shape
    N = slots.shape[0]
    info = plsc.get_sparse_core_info()
    NW = info.num_cores * info.num_subcores
    chunk = N // NW
    mesh = plsc.VectorSubcoreMesh(core_axis_name="c", subcore_axis_name="s")

    @functools.partial(
        pl.kernel,
        mesh=mesh,
        out_type=jax.ShapeDtypeStruct((N, D), jnp.float32),
        scratch_types=[
            pltpu.VMEM((chunk,), jnp.int32),
            pltpu.VMEM((chunk, D), jnp.float32),
            pltpu.SemaphoreType.DMA,
        ],
    )
    def k(y_hbm, slots_hbm, out_hbm, idx_v, rows_v, sem):
        wid = lax.axis_index("s") * info.num_cores + lax.axis_index("c")
        jbase = wid * chunk
        pltpu.sync_copy(slots_hbm.at[pl.ds(jbase, chunk)], idx_v)
        pltpu.async_copy(y_hbm.at[idx_v], rows_v, sem).wait()
        pltpu.sync_copy(rows_v, out_hbm.at[pl.ds(jbase, chunk)])

    return k(y, slots)


def _ffn_body(eid_ref, flag_ref, xdi_ref, xd_ref, wg_ref, wu_ref, wd_ref,
              y_ref):
    i = pl.program_id(0)

    @pl.when(flag_ref[i] == 1)
    def _():
        xt = xd_ref[...]
        g = jnp.dot(xt, wg_ref[0], preferred_element_type=jnp.float32)
        u = jnp.dot(xt, wu_ref[0], preferred_element_type=jnp.float32)
        h = g * jax.nn.sigmoid(g) * u
        y_ref[...] = jnp.dot(h, wd_ref[0], preferred_element_type=jnp.float32)


def _grouped_ffn(eid, flag, xdi, xd, w_gate, w_up, w_down, NT):
    R, D = xd.shape
    FF = w_gate.shape[2]
    grid_spec = pltpu.PrefetchScalarGridSpec(
        num_scalar_prefetch=3,
        grid=(NT,),
        in_specs=[
            pl.BlockSpec((BLK, D), lambda i, e, f, xi: (xi[i], 0)),
            pl.BlockSpec((1, D, FF), lambda i, e, f, xi: (e[i], 0, 0)),
            pl.BlockSpec((1, D, FF), lambda i, e, f, xi: (e[i], 0, 0)),
            pl.BlockSpec((1, FF, D), lambda i, e, f, xi: (e[i], 0, 0)),
        ],
        out_specs=pl.BlockSpec((BLK, D), lambda i, e, f, xi: (xi[i], 0)),
    )
    return pl.pallas_call(
        _ffn_body,
        grid_spec=grid_spec,
        out_shape=jax.ShapeDtypeStruct((R, D), jnp.float32),
    )(eid, flag, xdi, xd, w_gate, w_up, w_down)


def _final_body(x_ref, y0_ref, y1_ref, w_ref, sg_ref, su_ref, sd_ref,
                segw_ref, o_ref):
    x = x_ref[...]
    g = jnp.dot(x, sg_ref[...], preferred_element_type=jnp.float32)
    u = jnp.dot(x, su_ref[...], preferred_element_type=jnp.float32)
    h = g * jax.nn.sigmoid(g) * u
    sh = jnp.dot(h, sd_ref[...], preferred_element_type=jnp.float32)
    gate = jax.nn.sigmoid(jnp.dot(x, segw_ref[...],
                                  preferred_element_type=jnp.float32))[:, 0:1]
    w0 = w_ref[:, 0:1]
    w1 = w_ref[:, 1:2]
    o_ref[...] = w0 * y0_ref[...] + w1 * y1_ref[...] + gate * sh


def _final(x, y0, y1, wts, s_gate, s_up, s_down, segw_p):
    T, D = x.shape
    FFS = s_gate.shape[1]
    E = wts.shape[1]
    TB = 1024
    return pl.pallas_call(
        _final_body,
        grid=(T // TB,),
        in_specs=[
            pl.BlockSpec((TB, D), lambda i: (i, 0)),
            pl.BlockSpec((TB, D), lambda i: (i, 0)),
            pl.BlockSpec((TB, D), lambda i: (i, 0)),
            pl.BlockSpec((TB, E), lambda i: (i, 0)),
            pl.BlockSpec((D, FFS), lambda i: (0, 0)),
            pl.BlockSpec((D, FFS), lambda i: (0, 0)),
            pl.BlockSpec((FFS, D), lambda i: (0, 0)),
            pl.BlockSpec((D, 128), lambda i: (0, 0)),
        ],
        out_specs=pl.BlockSpec((TB, D), lambda i: (i, 0)),
        out_shape=jax.ShapeDtypeStruct((T, D), jnp.float32),
    )(x, y0, y1, wts, s_gate, s_up, s_down, segw_p)


def kernel(hidden_states, gate_w, w_gate, w_up, w_down, s_gate, s_up, s_down,
           seg_w):
    orig_shape = hidden_states.shape
    D = orig_shape[-1]
    x = hidden_states.reshape(-1, D)
    T = x.shape[0]
    E = w_gate.shape[0]
    NT = (T * TOPK) // BLK + E
    n_rows = NT * BLK

    slots2, wts, meta = _routing(x, gate_w, NT)
    slots = jnp.concatenate([slots2[:, 0], slots2[:, 1]])
    eid = meta[:, 0]
    flag = meta[:, 1]
    xdi = meta[:, 2]

    xd = _sc_dispatch(x, slots, n_rows)
    y = _grouped_ffn(eid, flag, xdi, xd, w_gate, w_up, w_down, NT)
    yg = _sc_combine(y, slots)
    y0 = yg[:T]
    y1 = yg[T:]

    segw_p = jnp.pad(seg_w, ((0, 0), (0, 128 - seg_w.shape[1])))
    out = _final(x, y0, y1, wts, s_gate, s_up, s_down, segw_p)
    return out.reshape(orig_shape)


# speedup vs baseline: 1.2863x; 1.1561x over previous
import functools

import jax
import jax.numpy as jnp
from jax import lax
from jax.experimental import pallas as pl
from jax.experimental.pallas import tpu as pltpu
from jax.experimental.pallas import tpu_sc as plsc

TOPK = 2
BLK = 576


def _pack_bf16(x):
    h = x.shape[1] // 2
    a = x[:, :h].astype(jnp.bfloat16).astype(jnp.float32)
    b = x[:, h:].astype(jnp.bfloat16).astype(jnp.float32)
    a32 = lax.bitcast_convert_type(a, jnp.int32)
    b32 = lax.bitcast_convert_type(b, jnp.int32)
    return a32 | lax.shift_right_logical(b32, 16)


def _unpack_bf16(p):
    a32 = p & jnp.int32(-65536)
    b32 = p << 16
    a = lax.bitcast_convert_type(a32, jnp.float32)
    b = lax.bitcast_convert_type(b32, jnp.float32)
    return jnp.concatenate([a, b], axis=1)


def _routing_body(x_ref, gw_ref, p_ref, w_ref, meta_ref, xb_ref, *, T, E, NT):
    x = x_ref[...]
    xb_ref[...] = _pack_bf16(x)
    gw = gw_ref[...]
    logits = lax.dot_general(x, gw, (((1,), (1,)), ((), ())),
                             preferred_element_type=jnp.float32)
    m = jnp.max(logits, axis=1, keepdims=True)
    ex = jnp.exp(logits - m)
    probs = ex / jnp.sum(ex, axis=1, keepdims=True)

    iota_e = lax.broadcasted_iota(jnp.int32, (T, E), 1)
    v0 = jnp.max(probs, axis=1, keepdims=True)
    i0 = jnp.min(jnp.where(probs == v0, iota_e, E), axis=1, keepdims=True)
    sel0 = iota_e == i0
    probs1 = jnp.where(sel0, -1.0, probs)
    v1 = jnp.max(probs1, axis=1, keepdims=True)
    i1 = jnp.min(jnp.where(probs1 == v1, iota_e, E), axis=1, keepdims=True)
    sel1 = iota_e == i1
    s = v0 + v1
    w0 = v0 / s
    w1 = v1 / s

    N = TOPK * T
    A = jnp.concatenate([sel0, sel1], axis=0).astype(jnp.float32)
    At = jnp.transpose(A, (1, 0))
    Ct = At
    sft = 1
    while sft < N:
        Ct = Ct + jnp.concatenate(
            [jnp.zeros((E, sft), jnp.float32), Ct[:, : N - sft]], axis=1)
        sft *= 2
    counts_c = Ct[:, N - 1: N]
    fblk = jnp.float32(BLK)
    pc_c = jnp.floor((counts_c + (fblk - 1.0)) * (1.0 / fblk)) * fblk
    ii = lax.broadcasted_iota(jnp.int32, (E, E), 0)
    ee = lax.broadcasted_iota(jnp.int32, (E, E), 1)
    lower_t = (ee < ii).astype(jnp.float32)
    base_c = lax.dot_general(lower_t, pc_c, (((1,), (0,)), ((), ())))
    slot_t = jnp.sum(At * (Ct - 1.0 + base_c), axis=0, keepdims=True)
    slot_t = slot_t.astype(jnp.int32)
    s0 = jnp.transpose(slot_t[:, :T], (1, 0))
    s1 = jnp.transpose(slot_t[:, T:], (1, 0))

    col = lax.broadcasted_iota(jnp.int32, (T, E), 1)
    p_ref[...] = jnp.where(col == 0, s0, jnp.where(col == 1, s1, 0))
    w_ref[...] = jnp.where(col == 0, w0, jnp.where(col == 1, w1, 0.0))

    base = jnp.transpose(base_c, (1, 0))
    pc = jnp.transpose(pc_c, (1, 0))
    counts = jnp.transpose(counts_c, (1, 0))
    ti = lax.broadcasted_iota(jnp.int32, (NT, E), 0)
    te = lax.broadcasted_iota(jnp.int32, (NT, E), 1)
    row = jnp.float32(BLK) * ti.astype(jnp.float32)
    inside = (row >= base) & (row < base + pc)
    eid = jnp.sum(jnp.where(inside, te, 0), axis=1, keepdims=True)
    nt_used = jnp.sum(pc_c) * (1.0 / fblk)
    tif = ti[:, :1].astype(jnp.float32)
    flag = (tif < nt_used).astype(jnp.int32)
    emax = jnp.max(jnp.where(counts > 0.0, ee[:1, :], 0), axis=1, keepdims=True)
    eid = jnp.where(flag == 1, eid, emax)
    xdi = jnp.minimum(tif, nt_used - 1.0).astype(jnp.int32)
    mcol = lax.broadcasted_iota(jnp.int32, (NT, E), 1)
    meta_ref[...] = jnp.where(
        mcol == 0, eid,
        jnp.where(mcol == 1, flag, jnp.where(mcol == 2, xdi, 0)))


def _routing(x, gate_w, NT):
    T, D = x.shape
    E = gate_w.shape[0]
    body = functools.partial(_routing_body, T=T, E=E, NT=NT)
    return pl.pallas_call(
        body,
        out_shape=(
            jax.ShapeDtypeStruct((T, E), jnp.int32),
            jax.ShapeDtypeStruct((T, E), jnp.float32),
            jax.ShapeDtypeStruct((NT, E), jnp.int32),
            jax.ShapeDtypeStruct((T, D // 2), jnp.int32),
        ),
    )(x, gate_w)


def _sc_dispatch(x, slots, n_rows):
    T, D = x.shape
    N = slots.shape[0]
    info = plsc.get_sparse_core_info()
    NW = info.num_cores * info.num_subcores
    chunk = N // NW
    mesh = plsc.VectorSubcoreMesh(core_axis_name="c", subcore_axis_name="s")

    @functools.partial(
        pl.kernel,
        mesh=mesh,
        out_type=jax.ShapeDtypeStruct((n_rows, D), jnp.int32),
        scratch_types=[
            pltpu.VMEM((chunk,), jnp.int32),
            pltpu.VMEM((chunk, D), jnp.int32),
            pltpu.SemaphoreType.DMA,
        ],
    )
    def k(x_hbm, slots_hbm, out_hbm, idx_v, rows_v, sem):
        wid = lax.axis_index("s") * info.num_cores + lax.axis_index("c")
        jbase = wid * chunk
        tbase = jnp.where(jbase >= T, jbase - T, jbase)
        pltpu.sync_copy(slots_hbm.at[pl.ds(jbase, chunk)], idx_v)
        pltpu.sync_copy(x_hbm.at[pl.ds(tbase, chunk)], rows_v)
        pltpu.async_copy(rows_v, out_hbm.at[idx_v], sem).wait()

    return k(x, slots)


def _sc_combine(y, slots):
    R, D = y.shape
    N = slots.shape[0]
    info = plsc.get_sparse_core_info()
    NW = info.num_cores * info.num_subcores
    chunk = N // NW
    mesh = plsc.VectorSubcoreMesh(core_axis_name="c", subcore_axis_name="s")

    @functools.partial(
        pl.kernel,
        mesh=mesh,
        out_type=jax.ShapeDtypeStruct((N, D), jnp.int32),
        scratch_types=[
            pltpu.VMEM((chunk,), jnp.int32),
            pltpu.VMEM((chunk, D), jnp.int32),
            pltpu.SemaphoreType.DMA,
        ],
    )
    def k(y_hbm, slots_hbm, out_hbm, idx_v, rows_v, sem):
        wid = lax.axis_index("s") * info.num_cores + lax.axis_index("c")
        jbase = wid * chunk
        pltpu.sync_copy(slots_hbm.at[pl.ds(jbase, chunk)], idx_v)
        pltpu.async_copy(y_hbm.at[idx_v], rows_v, sem).wait()
        pltpu.sync_copy(rows_v, out_hbm.at[pl.ds(jbase, chunk)])

    return k(y, slots)


def _ffn_body(eid_ref, flag_ref, xdi_ref, xd_ref, wg_ref, wu_ref, wd_ref,
              y_ref):
    i = pl.program_id(0)

    @pl.when(flag_ref[i] == 1)
    def _():
        xt = _unpack_bf16(xd_ref[...])
        g = jnp.dot(xt, wg_ref[0], preferred_element_type=jnp.float32)
        u = jnp.dot(xt, wu_ref[0], preferred_element_type=jnp.float32)
        h = g * jax.nn.sigmoid(g) * u
        y = jnp.dot(h, wd_ref[0], preferred_element_type=jnp.float32)
        y_ref[...] = _pack_bf16(y)


def _grouped_ffn(eid, flag, xdi, xd, w_gate, w_up, w_down, NT):
    R, Dp = xd.shape
    D = w_gate.shape[1]
    FF = w_gate.shape[2]
    grid_spec = pltpu.PrefetchScalarGridSpec(
        num_scalar_prefetch=3,
        grid=(NT,),
        in_specs=[
            pl.BlockSpec((BLK, Dp), lambda i, e, f, xi: (xi[i], 0)),
            pl.BlockSpec((1, D, FF), lambda i, e, f, xi: (e[i], 0, 0)),
            pl.BlockSpec((1, D, FF), lambda i, e, f, xi: (e[i], 0, 0)),
            pl.BlockSpec((1, FF, D), lambda i, e, f, xi: (e[i], 0, 0)),
        ],
        out_specs=pl.BlockSpec((BLK, Dp), lambda i, e, f, xi: (xi[i], 0)),
    )
    return pl.pallas_call(
        _ffn_body,
        grid_spec=grid_spec,
        out_shape=jax.ShapeDtypeStruct((R, Dp), jnp.int32),
    )(eid, flag, xdi, xd, w_gate, w_up, w_down)


def _final_body(x_ref, y0_ref, y1_ref, w_ref, sg_ref, su_ref, sd_ref,
                segw_ref, o_ref):
    x = x_ref[...]
    g = jnp.dot(x, sg_ref[...], preferred_element_type=jnp.float32)
    u = jnp.dot(x, su_ref[...], preferred_element_type=jnp.float32)
    h = g * jax.nn.sigmoid(g) * u
    sh = jnp.dot(h, sd_ref[...], preferred_element_type=jnp.float32)
    gate = jax.nn.sigmoid(jnp.dot(x, segw_ref[...],
                                  preferred_element_type=jnp.float32))[:, 0:1]
    w0 = w_ref[:, 0:1]
    w1 = w_ref[:, 1:2]
    y0 = _unpack_bf16(y0_ref[...])
    y1 = _unpack_bf16(y1_ref[...])
    o_ref[...] = w0 * y0 + w1 * y1 + gate * sh


def _final(x, y0, y1, wts, s_gate, s_up, s_down, segw_p):
    T, D = x.shape
    FFS = s_gate.shape[1]
    E = wts.shape[1]
    TB = 1024
    return pl.pallas_call(
        _final_body,
        grid=(T // TB,),
        in_specs=[
            pl.BlockSpec((TB, D), lambda i: (i, 0)),
            pl.BlockSpec((TB, D // 2), lambda i: (i, 0)),
            pl.BlockSpec((TB, D // 2), lambda i: (i, 0)),
            pl.BlockSpec((TB, E), lambda i: (i, 0)),
            pl.BlockSpec((D, FFS), lambda i: (0, 0)),
            pl.BlockSpec((D, FFS), lambda i: (0, 0)),
            pl.BlockSpec((FFS, D), lambda i: (0, 0)),
            pl.BlockSpec((D, 128), lambda i: (0, 0)),
        ],
        out_specs=pl.BlockSpec((TB, D), lambda i: (i, 0)),
        out_shape=jax.ShapeDtypeStruct((T, D), jnp.float32),
    )(x, y0, y1, wts, s_gate, s_up, s_down, segw_p)


def kernel(hidden_states, gate_w, w_gate, w_up, w_down, s_gate, s_up, s_down,
           seg_w):
    orig_shape = hidden_states.shape
    D = orig_shape[-1]
    x = hidden_states.reshape(-1, D)
    T = x.shape[0]
    E = w_gate.shape[0]
    NT = (T * TOPK) // BLK + E
    n_rows = NT * BLK

    slots2, wts, meta, xb = _routing(x, gate_w, NT)
    slots = jnp.concatenate([slots2[:, 0], slots2[:, 1]])
    eid = meta[:, 0]
    flag = meta[:, 1]
    xdi = meta[:, 2]

    xd = _sc_dispatch(xb, slots, n_rows)
    y = _grouped_ffn(eid, flag, xdi, xd, w_gate, w_up, w_down, NT)
    yg = _sc_combine(y, slots)
    y0 = yg[:T]
    y1 = yg[T:]

    segw_p = jnp.pad(seg_w, ((0, 0), (0, 128 - seg_w.shape[1])))
    out = _final(x, y0, y1, wts, s_gate, s_up, s_down, segw_p)
    return out.reshape(orig_shape)
